# baseline (speedup 1.0000x reference)
import jax
import jax.numpy as jnp
from jax import lax
from jax.experimental import pallas as pl

B, T, D = 64, 1024, 32
K = 1024
TB = 1024


def _vq_body(h_ref, cb_ref, cbm2_ref, c2_ref, iota_ref, q_ref, idx_ref):
    h = h_ref[0]
    nxc = lax.dot_general(cbm2_ref[...], h, (((1,), (1,)), ((), ())),
                          preferred_element_type=jnp.float32)
    dist = nxc + c2_ref[...]
    m = jnp.min(dist, axis=0, keepdims=True)
    hit = dist <= m
    idxf = jnp.min(jnp.where(hit, iota_ref[...], 65536.0),
                   axis=0, keepdims=True)
    idx_row = idxf.astype(jnp.int32)
    onehot = hit.astype(jnp.float32)
    q = lax.dot_general(onehot, cb_ref[...], (((0,), (0,)), ((), ())),
                        preferred_element_type=jnp.float32)
    q_ref[0] = h + (q - h)
    r = lax.rem(pl.program_id(0), 8)
    idx_ref[pl.ds(r, 1), :] = idx_row


@jax.jit
def kernel(h, codebook):
    bsz, t, d = h.shape
    grid = bsz * t // TB
    cbm2 = -2.0 * codebook
    c2 = lax.dot_general(codebook * codebook,
                         jnp.ones((d, 1), jnp.float32),
                         (((1,), (0,)), ((), ())))
    iota = lax.broadcasted_iota(jnp.float32, (K, 1), 0)
    q, idx = pl.pallas_call(
        _vq_body,
        grid=(grid,),
        in_specs=[
            pl.BlockSpec((1, TB, d), lambda i: (i, 0, 0)),
            pl.BlockSpec((K, d), lambda i: (0, 0)),
            pl.BlockSpec((K, d), lambda i: (0, 0)),
            pl.BlockSpec((K, 1), lambda i: (0, 0)),
            pl.BlockSpec((K, 1), lambda i: (0, 0)),
        ],
        out_specs=[
            pl.BlockSpec((1, TB, d), lambda i: (i, 0, 0)),
            pl.BlockSpec((8, T), lambda i: (i // 8, 0)),
        ],
        out_shape=[
            jax.ShapeDtypeStruct((bsz, t, d), jnp.float32),
            jax.ShapeDtypeStruct((bsz, t), jnp.int32),
        ],
    )(h, codebook, cbm2, c2, iota)
    return q, idx


# speedup vs baseline: 1.0563x; 1.0563x over previous
import jax
import jax.numpy as jnp
from jax import lax
from jax.experimental import pallas as pl
from jax.experimental.pallas import tpu as pltpu

B, T, D = 64, 1024, 32
K = 1024
TB = 1024


def _vq_body(h_ref, cbm2_ref, c2_ref, w2_ref, q_ref, idx_ref, acc_ref):
    h = h_ref[0]
    nxc = lax.dot_general(h, cbm2_ref[...], (((1,), (1,)), ((), ())),
                          preferred_element_type=jnp.float32)
    dist = nxc + c2_ref[...]
    m = jnp.min(dist, axis=1, keepdims=True)
    onehot = (dist <= m).astype(jnp.float32)
    qi = lax.dot_general(onehot, w2_ref[...], (((1,), (0,)), ((), ())),
                         preferred_element_type=jnp.float32)
    q = qi[:, :D]
    q_ref[0] = h + (q - h)
    r = lax.rem(pl.program_id(0), 8)
    lane = lax.broadcasted_iota(jnp.int32, (TB, 8), 1)
    acc_ref[...] = jnp.where(lane == r, qi[:, D:D + 1], acc_ref[...])

    @pl.when(r == 7)
    def _flush():
        idx_ref[...] = lax.transpose(acc_ref[...], (1, 0)).astype(jnp.int32)


@jax.jit
def kernel(h, codebook):
    bsz, t, d = h.shape
    grid = bsz * t // TB
    cbm2 = -2.0 * codebook
    c2 = jnp.sum(codebook ** 2, axis=1)[None, :]
    w2 = jnp.concatenate(
        [codebook, lax.broadcasted_iota(jnp.float32, (K, 1), 0)], axis=1)
    q, idx = pl.pallas_call(
        _vq_body,
        grid=(grid,),
        in_specs=[
            pl.BlockSpec((1, TB, d), lambda i: (i, 0, 0)),
            pl.BlockSpec((K, d), lambda i: (0, 0)),
            pl.BlockSpec((1, K), lambda i: (0, 0)),
            pl.BlockSpec((K, d + 1), lambda i: (0, 0)),
        ],
        out_specs=[
            pl.BlockSpec((1, TB, d), lambda i: (i, 0, 0)),
            pl.BlockSpec((8, T), lambda i: (i // 8, 0)),
        ],
        out_shape=[
            jax.ShapeDtypeStruct((bsz, t, d), jnp.float32),
            jax.ShapeDtypeStruct((bsz, t), jnp.int32),
        ],
        scratch_shapes=[pltpu.VMEM((TB, 8), jnp.float32)],
    )(h, cbm2, c2, w2)
    return q, idx


# speedup vs baseline: 1.4519x; 1.3746x over previous
import jax
import jax.numpy as jnp
from jax import lax
from jax.experimental import pallas as pl

B, T, D = 64, 1024, 32
K = 1024
TB = 4096


def _vq_body(h_ref, w1_ref, c2_ref, w2_ref, q_ref, idx_ref):
    h = h_ref[...]
    nxc = lax.dot_general(h, w1_ref[...], (((1,), (0,)), ((), ())),
                          preferred_element_type=jnp.float32)
    dist = nxc + c2_ref[...]
    m = jnp.min(dist, axis=1, keepdims=True)
    onehot = (dist <= m).astype(jnp.float32)
    qi = lax.dot_general(onehot, w2_ref[...], (((1,), (0,)), ((), ())),
                         preferred_element_type=jnp.float32)
    q = qi[:, :D]
    q_ref[...] = h + (q - h)
    idx_ref[...] = qi[:, D:D + 1].astype(jnp.int32)


@jax.jit
def kernel(h, codebook):
    bsz, t, d = h.shape
    n = bsz * t
    grid = n // TB
    flat = h.reshape(n, d)
    w1 = -2.0 * codebook.T
    c2 = jnp.sum(codebook ** 2, axis=1)[None, :]
    w2 = jnp.concatenate(
        [codebook, lax.broadcasted_iota(jnp.float32, (K, 1), 0)], axis=1)
    q_flat, idx_col = pl.pallas_call(
        _vq_body,
        grid=(grid,),
        in_specs=[
            pl.BlockSpec((TB, d), lambda i: (i, 0)),
            pl.BlockSpec((d, K), lambda i: (0, 0)),
            pl.BlockSpec((1, K), lambda i: (0, 0)),
            pl.BlockSpec((K, d + 1), lambda i: (0, 0)),
        ],
        out_specs=[
            pl.BlockSpec((TB, d), lambda i: (i, 0)),
            pl.BlockSpec((TB, 1), lambda i: (i, 0)),
        ],
        out_shape=[
            jax.ShapeDtypeStruct((n, d), jnp.float32),
            jax.ShapeDtypeStruct((n, 1), jnp.int32),
        ],
    )(flat, w1, c2, w2)
    return q_flat.reshape(bsz, t, d), idx_col.reshape(bsz, t)
